# baseline (device time: 30016 ns/iter reference)
import jax
import jax.numpy as jnp
from jax import lax
from jax.experimental import pallas as pl
from jax.experimental.pallas import tpu as pltpu

N_DEV = 32
BLK = 32


def kernel(x, w_mat):
    m_total, k_local = x.shape
    k_total, n = w_mat.shape

    def body(x_ref, w_ref, out_ref, xst_ref, xrt_ref, send_sems, recv_sems):
        me = lax.axis_index("i")

        for i in range(N_DEV):
            xst_ref[pl.ds(i * BLK, BLK), :] = x_ref[pl.ds(i * BLK, BLK), :].T

        rdmas = []
        for d in range(1, N_DEV):
            t = (me + d) % N_DEV
            rdma = pltpu.make_async_remote_copy(
                src_ref=xst_ref.at[pl.ds(t * BLK, BLK), :],
                dst_ref=xrt_ref.at[pl.ds(me * BLK, BLK), :],
                send_sem=send_sems.at[d],
                recv_sem=recv_sems.at[d],
                device_id=(t,),
                device_id_type=pl.DeviceIdType.MESH,
            )
            rdma.start()
            rdmas.append(rdma)

        xrt_ref[pl.ds(me * BLK, BLK), :] = xst_ref[pl.ds(me * BLK, BLK), :]

        for r in rdmas:
            r.wait_recv()
        out_ref[:, :] = lax.dot_general(
            xrt_ref[:, :],
            w_ref[:, :],
            dimension_numbers=(((0,), (0,)), ((), ())),
            preferred_element_type=jnp.float32,
        )
        for r in rdmas:
            r.wait_send()

    return pl.pallas_call(
        body,
        out_shape=jax.ShapeDtypeStruct((BLK, n), jnp.float32),
        in_specs=[
            pl.BlockSpec(memory_space=pltpu.VMEM),
            pl.BlockSpec(memory_space=pltpu.VMEM),
        ],
        out_specs=pl.BlockSpec(memory_space=pltpu.VMEM),
        scratch_shapes=[
            pltpu.VMEM((k_total, BLK), x.dtype),
            pltpu.VMEM((k_total, BLK), x.dtype),
            pltpu.SemaphoreType.DMA((N_DEV,)),
            pltpu.SemaphoreType.DMA((N_DEV,)),
        ],
    )(x, w_mat)


# device time: 13787 ns/iter; 2.1771x vs baseline; 2.1771x over previous
import jax
import jax.numpy as jnp
from jax import lax
from jax.experimental import pallas as pl
from jax.experimental.pallas import tpu as pltpu

N_DEV = 32
G = 4
S = 8
BLK = 32


def kernel(x, w_mat):
    m_total, k_local = x.shape
    k_total, n = w_mat.shape

    def body(
        x_ref, w_ref, out_ref,
        s1_ref,
        r1_ref,
        r2_ref,
        s1_send, s1_recv,
        s2_send, s2_recv,
        s2_ready,
    ):
        me = lax.axis_index("i")
        g = me // S
        lam = me % S

        barrier = pltpu.get_barrier_semaphore()
        for d in range(1, S):
            peer = g * S + (lam + d) % S
            pl.semaphore_signal(
                barrier, inc=1,
                device_id=(peer,), device_id_type=pl.DeviceIdType.MESH,
            )
        for e in range(1, G):
            peer = ((g + e) % G) * S + lam
            pl.semaphore_signal(
                s2_ready, inc=1,
                device_id=(peer,), device_id_type=pl.DeviceIdType.MESH,
            )

        for p in range(N_DEV):
            gp, lp = p // S, p % S
            s1_ref[lp, gp, :, :] = x_ref[pl.ds(p * BLK, BLK), :].T

        pl.semaphore_wait(barrier, S - 1)

        s1_rdmas = []
        for d in range(1, S):
            lp = (lam + d) % S
            rdma = pltpu.make_async_remote_copy(
                src_ref=s1_ref.at[lp],
                dst_ref=r1_ref.at[:, lam],
                send_sem=s1_send.at[d],
                recv_sem=s1_recv.at[d],
                device_id=(g * S + lp,),
                device_id_type=pl.DeviceIdType.MESH,
            )
            rdma.start()
            s1_rdmas.append(rdma)
        r1_ref[:, lam] = s1_ref[lam]
        for r in s1_rdmas:
            r.wait_recv()

        pl.semaphore_wait(s2_ready, G - 1)
        s2_rdmas = []
        for e in range(1, G):
            gp = (g + e) % G
            rdma = pltpu.make_async_remote_copy(
                src_ref=r1_ref.at[gp],
                dst_ref=r2_ref.at[g],
                send_sem=s2_send.at[e],
                recv_sem=s2_recv.at[e],
                device_id=(gp * S + lam,),
                device_id_type=pl.DeviceIdType.MESH,
            )
            rdma.start()
            s2_rdmas.append(rdma)
        r2_ref[g] = r1_ref[g]
        for r in s2_rdmas:
            r.wait_recv()

        acc = None
        for q in range(G):
            xq = jnp.reshape(r2_ref[q], (S * BLK, BLK))
            part = lax.dot_general(
                xq,
                w_ref[pl.ds(q * S * BLK, S * BLK), :],
                dimension_numbers=(((0,), (0,)), ((), ())),
                preferred_element_type=jnp.float32,
            )
            acc = part if acc is None else acc + part
        out_ref[:, :] = acc

        for r in s1_rdmas:
            r.wait_send()
        for r in s2_rdmas:
            r.wait_send()

    return pl.pallas_call(
        body,
        out_shape=jax.ShapeDtypeStruct((BLK, n), jnp.float32),
        in_specs=[
            pl.BlockSpec(memory_space=pltpu.VMEM),
            pl.BlockSpec(memory_space=pltpu.VMEM),
        ],
        out_specs=pl.BlockSpec(memory_space=pltpu.VMEM),
        scratch_shapes=[
            pltpu.VMEM((S, G, BLK, BLK), x.dtype),
            pltpu.VMEM((G, S, BLK, BLK), x.dtype),
            pltpu.VMEM((G, S, BLK, BLK), x.dtype),
            pltpu.SemaphoreType.DMA((S,)),
            pltpu.SemaphoreType.DMA((S,)),
            pltpu.SemaphoreType.DMA((G,)),
            pltpu.SemaphoreType.DMA((G,)),
            pltpu.SemaphoreType.REGULAR,
        ],
        compiler_params=pltpu.CompilerParams(collective_id=0),
    )(x, w_mat)
